# baseline (speedup 1.0000x reference)
import jax
import jax.numpy as jnp
from jax.experimental import pallas as pl
from jax.experimental.pallas import tpu as pltpu

_BLOCK_ROWS = 32


def _copy_body(x_ref, o_ref):
    o_ref[...] = x_ref[...]


def kernel(x):
    b, s, d = x.shape
    x2 = x.reshape(s, d).astype(jnp.float32)
    grid = (s // _BLOCK_ROWS,)
    out = pl.pallas_call(
        _copy_body,
        grid=grid,
        in_specs=[pl.BlockSpec((_BLOCK_ROWS, d), lambda i: (i, 0))],
        out_specs=pl.BlockSpec((_BLOCK_ROWS, d), lambda i: (i, 0)),
        out_shape=jax.ShapeDtypeStruct((s, d), jnp.float32),
        compiler_params=pltpu.CompilerParams(
            dimension_semantics=("parallel",),
        ),
    )(x2)
    return out.reshape(b, s, d)


# speedup vs baseline: 1.0145x; 1.0145x over previous
import jax
import jax.numpy as jnp
from jax.experimental import pallas as pl
from jax.experimental.pallas import tpu as pltpu

_OUTER = 2
_INNER = 2


def _copy_body(x_ref, o_ref):
    o_ref[...] = x_ref[...]


def kernel(x):
    b, s, d = x.shape
    x2 = x.reshape(s, d).astype(jnp.float32)
    rows = s // (_OUTER * _INNER)
    out = pl.pallas_call(
        _copy_body,
        grid=(_OUTER, _INNER),
        in_specs=[pl.BlockSpec((rows, d), lambda i, j: (i * _INNER + j, 0))],
        out_specs=pl.BlockSpec((rows, d), lambda i, j: (i * _INNER + j, 0)),
        out_shape=jax.ShapeDtypeStruct((s, d), jnp.float32),
        compiler_params=pltpu.CompilerParams(
            dimension_semantics=("parallel", "arbitrary"),
        ),
    )(x2)
    return out.reshape(b, s, d)
